# baseline (speedup 1.0000x reference)
import jax
import jax.numpy as jnp
from jax import lax
from jax.experimental import pallas as pl
from jax.experimental.pallas import tpu as pltpu
from jax.experimental.pallas import tpu_sc as plsc

VOCAB = 1000000
EMBED_DIM = 64
PAD_DIM = 128
BATCH = 16384
SEQ = 200

NC = 2
NS = 16
NW = NC * NS
LANES = 16

ROWS_PER_W = BATCH // NW
NB = 2
IDX_PER_CHUNK = NB * SEQ
STREAM = 100
NSTREAMS = IDX_PER_CHUNK // STREAM
NCHUNKS = ROWS_PER_W // NB
NPAIRS = NCHUNKS // 2
SCALE = 1.0 / SEQ


def _encoder_body(dseq, table, out, idx0, idx1, rows0, rows1, outst,
                  sem0, sem1):
    wid = lax.axis_index("s") * NC + lax.axis_index("c")
    base_row = wid * ROWS_PER_W

    idx_bufs = (idx0, idx1)
    row_bufs = (rows0, rows1)
    sems = (sem0, sem1)

    def fire(c, buf):
        ib, rb, sem = idx_bufs[buf], row_bufs[buf], sems[buf]
        irow0 = (base_row + c * NB) * (SEQ // STREAM)
        pltpu.sync_copy(dseq.at[pl.ds(irow0, NSTREAMS)], ib)
        for j in range(NSTREAMS):
            pltpu.async_copy(table.at[ib.at[j]],
                             rb.at[pl.ds(j * STREAM, STREAM)], sem)

    def drain(buf):
        rb, sem = row_bufs[buf], sems[buf]
        pltpu.make_async_copy(table.at[pl.ds(0, IDX_PER_CHUNK)], rb, sem).wait()

    def compute(c, buf):
        rb = row_bufs[buf]
        for b in range(NB):
            rbase = b * SEQ

            def body(j, accs):
                return tuple(
                    acc + rb[rbase + j, pl.ds(k * LANES, LANES)]
                    for k, acc in enumerate(accs)
                )

            zero = jnp.zeros((LANES,), jnp.float32)
            accs = lax.fori_loop(0, SEQ, body, (zero,) * (EMBED_DIM // LANES),
                                 unroll=4)
            for k, acc in enumerate(accs):
                outst[b, pl.ds(k * LANES, LANES)] = acc * SCALE
        pltpu.sync_copy(outst, out.at[pl.ds(base_row + c * NB, NB)])

    fire(0, 0)

    def pair(p, _):
        c0 = 2 * p
        fire(c0 + 1, 1)
        drain(0)
        compute(c0, 0)

        @pl.when(p + 1 < NPAIRS)
        def _():
            fire(c0 + 2, 0)

        drain(1)
        compute(c0 + 1, 1)
        return ()

    lax.fori_loop(0, NPAIRS, pair, ())


@jax.jit
def kernel(drug_seq, emb_table):
    tpad = jnp.pad(emb_table, ((0, 0), (0, PAD_DIM - EMBED_DIM)))
    dseq = drug_seq.reshape(BATCH * (SEQ // STREAM), STREAM).astype(jnp.int32)
    mesh = plsc.VectorSubcoreMesh(core_axis_name="c", subcore_axis_name="s")
    f = pl.kernel(
        _encoder_body,
        out_type=jax.ShapeDtypeStruct((BATCH, EMBED_DIM), jnp.float32),
        mesh=mesh,
        scratch_types=[
            pltpu.VMEM((NSTREAMS, STREAM), jnp.int32),
            pltpu.VMEM((NSTREAMS, STREAM), jnp.int32),
            pltpu.VMEM((IDX_PER_CHUNK, PAD_DIM), jnp.float32),
            pltpu.VMEM((IDX_PER_CHUNK, PAD_DIM), jnp.float32),
            pltpu.VMEM((NB, EMBED_DIM), jnp.float32),
            pltpu.SemaphoreType.DMA,
            pltpu.SemaphoreType.DMA,
        ],
        compiler_params=pltpu.CompilerParams(use_tc_tiling_on_sc=True),
    )
    return f(dseq, tpad)


# speedup vs baseline: 1.2420x; 1.2420x over previous
import jax
import jax.numpy as jnp
from jax import lax
from jax.experimental import pallas as pl
from jax.experimental.pallas import tpu as pltpu
from jax.experimental.pallas import tpu_sc as plsc

VOCAB = 1000000
EMBED_DIM = 64
PAD_DIM = 128
BATCH = 16384
SEQ = 200

NC = 2
NS = 16
NW = NC * NS
LANES = 16

ROWS_PER_W = BATCH // NW
NB = 4
IDX_PER_CHUNK = NB * SEQ
STREAM = 100
NSTREAMS = IDX_PER_CHUNK // STREAM
NCHUNKS = ROWS_PER_W // NB
NPAIRS = NCHUNKS // 2
SCALE = 1.0 / SEQ


def _encoder_body(dseq, table, out, idx0, idx1, rows0, rows1, outst,
                  sem0, sem1):
    wid = lax.axis_index("s") * NC + lax.axis_index("c")
    base_row = wid * ROWS_PER_W

    idx_bufs = (idx0, idx1)
    row_bufs = (rows0, rows1)
    sems = (sem0, sem1)

    def fire(c, buf):
        ib, rb, sem = idx_bufs[buf], row_bufs[buf], sems[buf]
        irow0 = (base_row + c * NB) * (SEQ // STREAM)
        pltpu.sync_copy(dseq.at[pl.ds(irow0, NSTREAMS)], ib)
        for j in range(NSTREAMS):
            pltpu.async_copy(table.at[ib.at[j]],
                             rb.at[pl.ds(j * STREAM, STREAM)], sem)

    def drain(buf):
        rb, sem = row_bufs[buf], sems[buf]
        pltpu.make_async_copy(table.at[pl.ds(0, IDX_PER_CHUNK)], rb, sem).wait()

    def compute(c, buf):
        rb = row_bufs[buf]
        for b in range(NB):
            rbase = b * SEQ

            def body(j, accs):
                return tuple(
                    acc + rb[rbase + j, pl.ds(k * LANES, LANES)]
                    for k, acc in enumerate(accs)
                )

            zero = jnp.zeros((LANES,), jnp.float32)
            accs = lax.fori_loop(0, SEQ, body, (zero,) * (EMBED_DIM // LANES),
                                 unroll=4)
            for k, acc in enumerate(accs):
                outst[b, pl.ds(k * LANES, LANES)] = acc * SCALE
        pltpu.sync_copy(outst, out.at[pl.ds(base_row + c * NB, NB)])

    fire(0, 0)

    def pair(p, _):
        c0 = 2 * p
        fire(c0 + 1, 1)
        drain(0)
        compute(c0, 0)

        @pl.when(p + 1 < NPAIRS)
        def _():
            fire(c0 + 2, 0)

        drain(1)
        compute(c0 + 1, 1)
        return ()

    lax.fori_loop(0, NPAIRS, pair, ())


@jax.jit
def kernel(drug_seq, emb_table):
    tpad = jnp.pad(emb_table, ((0, 0), (0, PAD_DIM - EMBED_DIM)))
    t2 = tpad.reshape(2 * VOCAB, EMBED_DIM)
    dseq = (drug_seq.astype(jnp.int32) * 2).reshape(
        BATCH * (SEQ // STREAM), STREAM)
    mesh = plsc.VectorSubcoreMesh(core_axis_name="c", subcore_axis_name="s")
    f = pl.kernel(
        _encoder_body,
        out_type=jax.ShapeDtypeStruct((BATCH, EMBED_DIM), jnp.float32),
        mesh=mesh,
        scratch_types=[
            pltpu.VMEM((NSTREAMS, STREAM), jnp.int32),
            pltpu.VMEM((NSTREAMS, STREAM), jnp.int32),
            pltpu.VMEM((IDX_PER_CHUNK, EMBED_DIM), jnp.float32),
            pltpu.VMEM((IDX_PER_CHUNK, EMBED_DIM), jnp.float32),
            pltpu.VMEM((NB, EMBED_DIM), jnp.float32),
            pltpu.SemaphoreType.DMA,
            pltpu.SemaphoreType.DMA,
        ],
        compiler_params=pltpu.CompilerParams(use_tc_tiling_on_sc=False),
    )
    return f(dseq, t2)
